# baseline (speedup 1.0000x reference)
import functools

import jax
import jax.numpy as jnp
from jax import lax
from jax.experimental import pallas as pl
from jax.experimental.pallas import tpu as pltpu
from jax.experimental.pallas import tpu_sc as plsc

L = 16
NC, NS = 2, 16
NW = NC * NS
K = 128
BLK = 1024


def _mesh():
  return plsc.VectorSubcoreMesh(core_axis_name="c", subcore_axis_name="s")


def _rsqrt16(x):
  i = plsc.bitcast(x, jnp.int32)
  y = plsc.bitcast(jnp.int32(0x5F3759DF) - (i >> 1), jnp.float32)
  for _ in range(3):
    y = y * (1.5 - 0.5 * x * y * y)
  return y


def _deg_body(ept, dst_hbm, zeros_hbm, dinv_hbm, dst_v, hist_v, iota_v, deg_t,
              deg_s):
  cid = lax.axis_index("c")
  sid = lax.axis_index("s")
  sl = pl.ds(sid * 8, 8)
  iota16 = lax.iota(jnp.int32, L)
  for k in range(128 // L):
    iota_v[0, pl.ds(k * L, L)] = iota16 + k * L

  @pl.when(cid == 0)
  def _():
    pltpu.sync_copy(zeros_hbm, hist_v)
    pltpu.sync_copy(zeros_hbm.at[pl.ds(0, 8)], deg_s.at[sl])
    pltpu.sync_copy(dst_hbm.at[pl.ds(sid * ept, ept)], dst_v)
    ones = jnp.full((L,), 1.0, jnp.float32)

    def body(i, carry):
      for u in range(4):
        idx = dst_v[pl.ds((i * 4 + u) * L, L)]
        plsc.addupdate_scatter(hist_v, [idx >> 7, idx & 127], ones)
      return carry

    lax.fori_loop(0, ept // (L * 4), body, 0)

  plsc.subcore_barrier()

  @pl.when(cid == 0)
  def _():
    pltpu.sync_copy(hist_v, deg_s.at[iota_v.at[0]], add=True)

  plsc.subcore_barrier()

  @pl.when(cid == 0)
  def _():
    pltpu.sync_copy(deg_s.at[sl], deg_t)
    for r in range(8):
      for k in range(128 // L):
        cs = pl.ds(k * L, L)
        deg_t[r, cs] = _rsqrt16(1.0 + deg_t[r, cs])
    pltpu.sync_copy(deg_t, dinv_hbm.at[sl])


def _make_deg_kernel(ept):
  return functools.partial(
      pl.kernel,
      out_type=jax.ShapeDtypeStruct((128, 128), jnp.float32),
      mesh=_mesh(),
      compiler_params=pltpu.CompilerParams(needs_layout_passes=False),
      scratch_types=[
          pltpu.VMEM((ept,), jnp.int32),
          pltpu.VMEM((128, 128), jnp.float32),
          pltpu.VMEM((1, 128), jnp.int32),
          pltpu.VMEM((8, 128), jnp.float32),
          pltpu.VMEM_SHARED((128, 128), jnp.float32),
      ],
  )(functools.partial(_deg_body, ept))


def _msg_body(chunks, rows_per_tile, nbuf, nsplit, iphases, w, *refs):
  h_list = refs[:nsplit]
  (src_hbm, dst_hbm, zeros_hbm, out_hbm,
   src_v, dst_v, rows_v, sems, hs, acc) = refs[nsplit:]
  cid = lax.axis_index("c")
  sid = lax.axis_index("s")
  wid = sid * NC + cid
  tbase = sid * rows_per_tile
  cph = chunks // iphases

  for p in range(nsplit):
    for z in range(rows_per_tile // K):
      pltpu.sync_copy(zeros_hbm, acc.at[pl.ds(tbase + z * K, K)])
    pltpu.sync_copy(h_list[p].at[pl.ds(tbase, rows_per_tile)],
                    hs.at[pl.ds(tbase, rows_per_tile)])
    plsc.subcore_barrier()

    for ip in range(iphases):
      cbase = wid * chunks + ip * cph
      pltpu.sync_copy(src_hbm.at[pl.ds(cbase, cph)], src_v)
      pltpu.sync_copy(dst_hbm.at[pl.ds(cbase, cph)], dst_v)
      for b in range(nbuf):
        pltpu.async_copy(hs.at[src_v.at[b]], rows_v.at[b], sems.at[b])

      def group(g, carry):
        for b in range(nbuf):
          j = g * nbuf + b
          pltpu.make_async_copy(hs.at[src_v.at[j]], rows_v.at[b],
                                sems.at[b]).wait()
          pltpu.sync_copy(rows_v.at[b], acc.at[dst_v.at[j]], add=True)
          jn = j + nbuf

          @pl.when(jn < cph)
          def _():
            pltpu.async_copy(hs.at[src_v.at[jn]], rows_v.at[b], sems.at[b])
        return carry

      lax.fori_loop(0, cph // nbuf, group, 0)
    plsc.subcore_barrier()
    base = (p * NC + cid) * (rows_per_tile * NS) + tbase
    pltpu.sync_copy(acc.at[pl.ds(tbase, rows_per_tile)],
                    out_hbm.at[pl.ds(base, rows_per_tile)])


def _make_msg_kernel(npad, d, chunks, nbuf, nsplit, iphases):
  rows_per_tile = npad // NS
  w = d // nsplit
  return functools.partial(
      pl.kernel,
      out_type=jax.ShapeDtypeStruct((nsplit * NC * npad, w), jnp.float32),
      mesh=_mesh(),
      compiler_params=pltpu.CompilerParams(
          needs_layout_passes=False, use_tc_tiling_on_sc=False),
      scratch_types=[
          pltpu.VMEM((chunks // iphases, K), jnp.int32),
          pltpu.VMEM((chunks // iphases, K), jnp.int32),
          pltpu.VMEM((nbuf, K, w), jnp.float32),
          pltpu.SemaphoreType.DMA((nbuf,)),
          pltpu.VMEM_SHARED((npad, w), jnp.float32),
          pltpu.VMEM_SHARED((npad, w), jnp.float32),
      ],
  )(functools.partial(_msg_body, chunks, rows_per_tile, nbuf, nsplit, iphases,
                      w))


def _mm_scale_body(hw, x_ref, w_ref, dinv_ref, outa_ref, outb_ref):
  h1 = jnp.dot(dinv_ref[...] * x_ref[...], w_ref[...],
               preferred_element_type=jnp.float32)
  outa_ref[...] = h1[:, :hw]
  outb_ref[...] = h1[:, hw:]


def _layer2_body(hw, p00_ref, p01_ref, p10_ref, p11_ref, h1a_ref, h1b_ref,
                 dinv_ref, b1_ref, w2_ref, out_ref):
  dinv = dinv_ref[...]
  b1 = b1_ref[...]
  h0 = jnp.maximum(
      dinv * (p00_ref[...] + p01_ref[...] + h1a_ref[...]) + b1[:, :hw], 0.0)
  h1 = jnp.maximum(
      dinv * (p10_ref[...] + p11_ref[...] + h1b_ref[...]) + b1[:, hw:], 0.0)
  mm = (jnp.dot(h0, w2_ref[:hw, :], preferred_element_type=jnp.float32) +
        jnp.dot(h1, w2_ref[hw:, :], preferred_element_type=jnp.float32))
  out_ref[...] = dinv * mm


def _final_body(c, p0_ref, p1_ref, h2_ref, dinv_ref, b2_ref, out_ref):
  agg = p0_ref[...] + p1_ref[...] + h2_ref[...]
  o = (dinv_ref[...] * agg + b2_ref[...])[:, :c]
  m = jnp.max(o, axis=1, keepdims=True)
  z = o - m
  lse = jnp.log(jnp.sum(jnp.exp(z), axis=1, keepdims=True))
  out_ref[...] = z - lse


def kernel(x, edge_index, W1, b1, W2, b2):
  n, f_in = x.shape
  hidden = W1.shape[1]
  c = W2.shape[1]
  e = edge_index.shape[1]

  npad = ((n + 2 * BLK) // (2 * BLK)) * (2 * BLK)
  chunks = -(-e // (NW * K))
  chunks = ((chunks + 7) // 8) * 8
  epad = NW * chunks * K
  epw = epad // NW
  cpad = ((c + L - 1) // L) * L

  src = jnp.concatenate([edge_index[0], jnp.zeros((epad - e,), jnp.int32)])
  dst = jnp.concatenate([edge_index[1],
                         jnp.full((epad - e,), n, jnp.int32)])
  src2d = src.reshape(epad // K, K)
  dst2d = dst.reshape(epad // K, K)
  xp = jnp.zeros((npad, f_in), x.dtype).at[:n].set(x)
  w2p = jnp.zeros((f_in, cpad), W2.dtype).at[:, :c].set(W2)
  b2p = jnp.zeros((cpad,), b2.dtype).at[:c].set(b2)

  dinv_sq = _make_deg_kernel(epad // NS)(dst, jnp.zeros((128, 128),
                                                        jnp.float32))
  dinv_col = dinv_sq.reshape(128 * 128)[:npad].reshape(npad, 1)

  grid = npad // BLK
  row_spec = pl.BlockSpec((BLK, 1), lambda i: (i, 0))

  w1h = hidden // 2
  h1a, h1b = pl.pallas_call(
      functools.partial(_mm_scale_body, w1h),
      grid=(grid,),
      in_specs=[
          pl.BlockSpec((BLK, f_in), lambda i: (i, 0)),
          pl.BlockSpec((f_in, hidden), lambda i: (0, 0)),
          row_spec,
      ],
      out_specs=[
          pl.BlockSpec((BLK, w1h), lambda i: (i, 0)),
          pl.BlockSpec((BLK, w1h), lambda i: (i, 0)),
      ],
      out_shape=[
          jax.ShapeDtypeStruct((npad, w1h), jnp.float32),
          jax.ShapeDtypeStruct((npad, w1h), jnp.float32),
      ],
  )(xp, W1, dinv_col)

  p1 = _make_msg_kernel(npad, hidden, chunks, 4, 2, 2)(
      h1a, h1b, src2d, dst2d, jnp.zeros((K, w1h), jnp.float32))

  def _reg(r, wd):
    return pl.BlockSpec((BLK, wd), lambda i, r=r: (r * grid + i, 0))

  h2 = pl.pallas_call(
      functools.partial(_layer2_body, w1h),
      grid=(grid,),
      in_specs=[
          _reg(0, w1h), _reg(1, w1h), _reg(2, w1h), _reg(3, w1h),
          pl.BlockSpec((BLK, w1h), lambda i: (i, 0)),
          pl.BlockSpec((BLK, w1h), lambda i: (i, 0)),
          row_spec,
          pl.BlockSpec((1, hidden), lambda i: (0, 0)),
          pl.BlockSpec((hidden, cpad), lambda i: (0, 0)),
      ],
      out_specs=pl.BlockSpec((BLK, cpad), lambda i: (i, 0)),
      out_shape=jax.ShapeDtypeStruct((npad, cpad), jnp.float32),
  )(p1, p1, p1, p1, h1a, h1b, dinv_col, b1.reshape(1, hidden), w2p)

  p2 = _make_msg_kernel(npad, cpad, chunks, 5, 1, 1)(
      h2, src2d, dst2d, jnp.zeros((K, cpad), jnp.float32))

  out = pl.pallas_call(
      functools.partial(_final_body, c),
      grid=(grid,),
      in_specs=[
          _reg(0, cpad), _reg(1, cpad),
          pl.BlockSpec((BLK, cpad), lambda i: (i, 0)),
          row_spec,
          pl.BlockSpec((1, cpad), lambda i: (0, 0)),
      ],
      out_specs=pl.BlockSpec((BLK, c), lambda i: (i, 0)),
      out_shape=jax.ShapeDtypeStruct((npad, c), jnp.float32),
  )(p2, p2, h2, dinv_col, b2p.reshape(1, cpad))

  return out[:n]


# speedup vs baseline: 1.0147x; 1.0147x over previous
import functools

import jax
import jax.numpy as jnp
from jax import lax
from jax.experimental import pallas as pl
from jax.experimental.pallas import tpu as pltpu
from jax.experimental.pallas import tpu_sc as plsc

L = 16
NC, NS = 2, 16
NW = NC * NS
K = 128
BLK = 1024


def _mesh():
  return plsc.VectorSubcoreMesh(core_axis_name="c", subcore_axis_name="s")


def _rsqrt16(x):
  i = plsc.bitcast(x, jnp.int32)
  y = plsc.bitcast(jnp.int32(0x5F3759DF) - (i >> 1), jnp.float32)
  for _ in range(3):
    y = y * (1.5 - 0.5 * x * y * y)
  return y


def _deg_body(ept, dst_hbm, zeros_hbm, dinv_hbm, dst_v, hist_v, iota_v, deg_t,
              deg_s):
  cid = lax.axis_index("c")
  sid = lax.axis_index("s")
  sl = pl.ds(sid * 8, 8)
  iota16 = lax.iota(jnp.int32, L)
  for k in range(128 // L):
    iota_v[0, pl.ds(k * L, L)] = iota16 + k * L

  @pl.when(cid == 0)
  def _():
    pltpu.sync_copy(zeros_hbm, hist_v)
    pltpu.sync_copy(zeros_hbm.at[pl.ds(0, 8)], deg_s.at[sl])
    pltpu.sync_copy(dst_hbm.at[pl.ds(sid * ept, ept)], dst_v)
    ones = jnp.full((L,), 1.0, jnp.float32)

    def body(i, carry):
      for u in range(4):
        idx = dst_v[pl.ds((i * 4 + u) * L, L)]
        plsc.addupdate_scatter(hist_v, [idx >> 7, idx & 127], ones)
      return carry

    lax.fori_loop(0, ept // (L * 4), body, 0)

  plsc.subcore_barrier()

  @pl.when(cid == 0)
  def _():
    pltpu.sync_copy(hist_v, deg_s.at[iota_v.at[0]], add=True)

  plsc.subcore_barrier()

  @pl.when(cid == 0)
  def _():
    pltpu.sync_copy(deg_s.at[sl], deg_t)
    for r in range(8):
      for k in range(128 // L):
        cs = pl.ds(k * L, L)
        deg_t[r, cs] = _rsqrt16(1.0 + deg_t[r, cs])
    pltpu.sync_copy(deg_t, dinv_hbm.at[sl])


def _make_deg_kernel(ept):
  return functools.partial(
      pl.kernel,
      out_type=jax.ShapeDtypeStruct((128, 128), jnp.float32),
      mesh=_mesh(),
      compiler_params=pltpu.CompilerParams(needs_layout_passes=False),
      scratch_types=[
          pltpu.VMEM((ept,), jnp.int32),
          pltpu.VMEM((128, 128), jnp.float32),
          pltpu.VMEM((1, 128), jnp.int32),
          pltpu.VMEM((8, 128), jnp.float32),
          pltpu.VMEM_SHARED((128, 128), jnp.float32),
      ],
  )(functools.partial(_deg_body, ept))


def _msg_body(chunks, rows_per_tile, nbuf, nsplit, iphases, w, *refs):
  h_list = refs[:nsplit]
  (src_hbm, dst_hbm, zeros_hbm, out_hbm,
   src_v, dst_v, rows_v, sems, hs, acc) = refs[nsplit:]
  cid = lax.axis_index("c")
  sid = lax.axis_index("s")
  wid = sid * NC + cid
  tbase = sid * rows_per_tile
  cph = chunks // iphases

  for p in range(nsplit):
    for z in range(rows_per_tile // K):
      pltpu.sync_copy(zeros_hbm, acc.at[pl.ds(tbase + z * K, K)])
    pltpu.sync_copy(h_list[p].at[pl.ds(tbase, rows_per_tile)],
                    hs.at[pl.ds(tbase, rows_per_tile)])
    plsc.subcore_barrier()

    for ip in range(iphases):
      cbase = wid * chunks + ip * cph
      pltpu.sync_copy(src_hbm.at[pl.ds(cbase, cph)], src_v)
      pltpu.sync_copy(dst_hbm.at[pl.ds(cbase, cph)], dst_v)
      for b in range(nbuf):
        pltpu.async_copy(hs.at[src_v.at[b]], rows_v.at[b], sems.at[b])

      def group(g, carry):
        for b in range(nbuf):
          j = g * nbuf + b
          pltpu.make_async_copy(hs.at[src_v.at[j]], rows_v.at[b],
                                sems.at[b]).wait()
          pltpu.sync_copy(rows_v.at[b], acc.at[dst_v.at[j]], add=True)
          jn = j + nbuf

          @pl.when(jn < cph)
          def _():
            pltpu.async_copy(hs.at[src_v.at[jn]], rows_v.at[b], sems.at[b])
        return carry

      lax.fori_loop(0, cph // nbuf, group, 0)
    plsc.subcore_barrier()
    base = (p * NC + cid) * (rows_per_tile * NS) + tbase
    pltpu.sync_copy(acc.at[pl.ds(tbase, rows_per_tile)],
                    out_hbm.at[pl.ds(base, rows_per_tile)])


def _make_msg_kernel(npad, d, chunks, nbuf, nsplit, iphases):
  rows_per_tile = npad // NS
  w = d // nsplit
  return functools.partial(
      pl.kernel,
      out_type=jax.ShapeDtypeStruct((nsplit * NC * npad, w), jnp.float32),
      mesh=_mesh(),
      compiler_params=pltpu.CompilerParams(
          needs_layout_passes=False, use_tc_tiling_on_sc=False),
      scratch_types=[
          pltpu.VMEM((chunks // iphases, K), jnp.int32),
          pltpu.VMEM((chunks // iphases, K), jnp.int32),
          pltpu.VMEM((nbuf, K, w), jnp.float32),
          pltpu.SemaphoreType.DMA((nbuf,)),
          pltpu.VMEM_SHARED((npad, w), jnp.float32),
          pltpu.VMEM_SHARED((npad, w), jnp.float32),
      ],
  )(functools.partial(_msg_body, chunks, rows_per_tile, nbuf, nsplit, iphases,
                      w))


def _mm_scale_body(hw, x_ref, w_ref, dinv_ref, outa_ref, outb_ref):
  h1 = jnp.dot(dinv_ref[...] * x_ref[...], w_ref[...],
               preferred_element_type=jnp.float32)
  outa_ref[...] = h1[:, :hw]
  outb_ref[...] = h1[:, hw:]


def _layer2_body(hw, p00_ref, p01_ref, p10_ref, p11_ref, h1a_ref, h1b_ref,
                 dinv_ref, b1_ref, w2_ref, out_ref):
  dinv = dinv_ref[...]
  b1 = b1_ref[...]
  h0 = jnp.maximum(
      dinv * (p00_ref[...] + p01_ref[...] + h1a_ref[...]) + b1[:, :hw], 0.0)
  h1 = jnp.maximum(
      dinv * (p10_ref[...] + p11_ref[...] + h1b_ref[...]) + b1[:, hw:], 0.0)
  mm = (jnp.dot(h0, w2_ref[:hw, :], preferred_element_type=jnp.float32) +
        jnp.dot(h1, w2_ref[hw:, :], preferred_element_type=jnp.float32))
  out_ref[...] = dinv * mm


def _final_body(c, p0_ref, p1_ref, h2_ref, dinv_ref, b2_ref, out_ref):
  agg = p0_ref[...] + p1_ref[...] + h2_ref[...]
  o = (dinv_ref[...] * agg + b2_ref[...])[:, :c]
  m = jnp.max(o, axis=1, keepdims=True)
  z = o - m
  lse = jnp.log(jnp.sum(jnp.exp(z), axis=1, keepdims=True))
  out_ref[...] = z - lse


def kernel(x, edge_index, W1, b1, W2, b2):
  n, f_in = x.shape
  hidden = W1.shape[1]
  c = W2.shape[1]
  e = edge_index.shape[1]

  npad = ((n + 2 * BLK) // (2 * BLK)) * (2 * BLK)
  chunks = -(-e // (NW * K))
  chunks = ((chunks + 7) // 8) * 8
  epad = NW * chunks * K
  epw = epad // NW
  cpad = ((c + L - 1) // L) * L

  src = jnp.concatenate([edge_index[0], jnp.zeros((epad - e,), jnp.int32)])
  dst = jnp.concatenate([edge_index[1],
                         jnp.full((epad - e,), n, jnp.int32)])
  src2d = src.reshape(epad // K, K)
  dst2d = dst.reshape(epad // K, K)
  xp = jnp.zeros((npad, f_in), x.dtype).at[:n].set(x)
  w2p = jnp.zeros((f_in, cpad), W2.dtype).at[:, :c].set(W2)
  b2p = jnp.zeros((cpad,), b2.dtype).at[:c].set(b2)

  dinv_sq = _make_deg_kernel(epad // NS)(dst, jnp.zeros((128, 128),
                                                        jnp.float32))
  dinv_col = dinv_sq.reshape(128 * 128)[:npad].reshape(npad, 1)

  grid = npad // BLK
  row_spec = pl.BlockSpec((BLK, 1), lambda i: (i, 0))

  w1h = hidden // 2
  h1a, h1b = pl.pallas_call(
      functools.partial(_mm_scale_body, w1h),
      grid=(grid,),
      in_specs=[
          pl.BlockSpec((BLK, f_in), lambda i: (i, 0)),
          pl.BlockSpec((f_in, hidden), lambda i: (0, 0)),
          row_spec,
      ],
      out_specs=[
          pl.BlockSpec((BLK, w1h), lambda i: (i, 0)),
          pl.BlockSpec((BLK, w1h), lambda i: (i, 0)),
      ],
      out_shape=[
          jax.ShapeDtypeStruct((npad, w1h), jnp.float32),
          jax.ShapeDtypeStruct((npad, w1h), jnp.float32),
      ],
  )(xp, W1, dinv_col)

  p1 = _make_msg_kernel(npad, hidden, chunks, 2, 2, 1)(
      h1a, h1b, src2d, dst2d, jnp.zeros((K, w1h), jnp.float32))

  def _reg(r, wd):
    return pl.BlockSpec((BLK, wd), lambda i, r=r: (r * grid + i, 0))

  h2 = pl.pallas_call(
      functools.partial(_layer2_body, w1h),
      grid=(grid,),
      in_specs=[
          _reg(0, w1h), _reg(1, w1h), _reg(2, w1h), _reg(3, w1h),
          pl.BlockSpec((BLK, w1h), lambda i: (i, 0)),
          pl.BlockSpec((BLK, w1h), lambda i: (i, 0)),
          row_spec,
          pl.BlockSpec((1, hidden), lambda i: (0, 0)),
          pl.BlockSpec((hidden, cpad), lambda i: (0, 0)),
      ],
      out_specs=pl.BlockSpec((BLK, cpad), lambda i: (i, 0)),
      out_shape=jax.ShapeDtypeStruct((npad, cpad), jnp.float32),
  )(p1, p1, p1, p1, h1a, h1b, dinv_col, b1.reshape(1, hidden), w2p)

  p2 = _make_msg_kernel(npad, cpad, chunks, 4, 1, 1)(
      h2, src2d, dst2d, jnp.zeros((K, cpad), jnp.float32))

  out = pl.pallas_call(
      functools.partial(_final_body, c),
      grid=(grid,),
      in_specs=[
          _reg(0, cpad), _reg(1, cpad),
          pl.BlockSpec((BLK, cpad), lambda i: (i, 0)),
          row_spec,
          pl.BlockSpec((1, cpad), lambda i: (0, 0)),
      ],
      out_specs=pl.BlockSpec((BLK, c), lambda i: (i, 0)),
      out_shape=jax.ShapeDtypeStruct((npad, c), jnp.float32),
  )(p2, p2, h2, dinv_col, b2p.reshape(1, cpad))

  return out[:n]


# speedup vs baseline: 1.2743x; 1.2558x over previous
import functools

import jax
import jax.numpy as jnp
from jax import lax
from jax.experimental import pallas as pl
from jax.experimental.pallas import tpu as pltpu
from jax.experimental.pallas import tpu_sc as plsc

L = 16
NC, NS = 2, 16
NW = NC * NS
K = 128
BLK = 1024


def _mesh():
  return plsc.VectorSubcoreMesh(core_axis_name="c", subcore_axis_name="s")


def _rsqrt16(x):
  i = plsc.bitcast(x, jnp.int32)
  y = plsc.bitcast(jnp.int32(0x5F3759DF) - (i >> 1), jnp.float32)
  for _ in range(3):
    y = y * (1.5 - 0.5 * x * y * y)
  return y


def _deg_body(ept, dst_hbm, zeros_hbm, dinv_hbm, dst_v, hist_v, iota_v, deg_t,
              deg_s):
  cid = lax.axis_index("c")
  sid = lax.axis_index("s")
  sl = pl.ds(sid * 8, 8)
  iota16 = lax.iota(jnp.int32, L)
  for k in range(128 // L):
    iota_v[0, pl.ds(k * L, L)] = iota16 + k * L

  @pl.when(cid == 0)
  def _():
    pltpu.sync_copy(zeros_hbm, hist_v)
    pltpu.sync_copy(zeros_hbm.at[pl.ds(0, 8)], deg_s.at[sl])
    pltpu.sync_copy(dst_hbm.at[pl.ds(sid * ept, ept)], dst_v)
    ones = jnp.full((L,), 1.0, jnp.float32)

    def body(i, carry):
      for u in range(4):
        idx = dst_v[pl.ds((i * 4 + u) * L, L)]
        plsc.addupdate_scatter(hist_v, [idx >> 7, idx & 127], ones)
      return carry

    lax.fori_loop(0, ept // (L * 4), body, 0)

  plsc.subcore_barrier()

  @pl.when(cid == 0)
  def _():
    pltpu.sync_copy(hist_v, deg_s.at[iota_v.at[0]], add=True)

  plsc.subcore_barrier()

  @pl.when(cid == 0)
  def _():
    pltpu.sync_copy(deg_s.at[sl], deg_t)
    for r in range(8):
      for k in range(128 // L):
        cs = pl.ds(k * L, L)
        deg_t[r, cs] = _rsqrt16(1.0 + deg_t[r, cs])
    pltpu.sync_copy(deg_t, dinv_hbm.at[sl])


def _make_deg_kernel(ept):
  return functools.partial(
      pl.kernel,
      out_type=jax.ShapeDtypeStruct((128, 128), jnp.float32),
      mesh=_mesh(),
      compiler_params=pltpu.CompilerParams(needs_layout_passes=False),
      scratch_types=[
          pltpu.VMEM((ept,), jnp.int32),
          pltpu.VMEM((128, 128), jnp.float32),
          pltpu.VMEM((1, 128), jnp.int32),
          pltpu.VMEM((8, 128), jnp.float32),
          pltpu.VMEM_SHARED((128, 128), jnp.float32),
      ],
  )(functools.partial(_deg_body, ept))


def _msg_body(chunks, rows_per_tile, nbuf, nsplit, iphases, w, *refs):
  h_list = refs[:nsplit]
  (src_hbm, dst_hbm, zeros_hbm, out_hbm,
   src_v, dst_v, rows_v, sems, hs, acc) = refs[nsplit:]
  cid = lax.axis_index("c")
  sid = lax.axis_index("s")
  wid = sid * NC + cid
  tbase = sid * rows_per_tile
  cph = chunks // iphases

  for p in range(nsplit):
    for z in range(rows_per_tile // K):
      pltpu.sync_copy(zeros_hbm, acc.at[pl.ds(tbase + z * K, K)])
    pltpu.sync_copy(h_list[p].at[pl.ds(tbase, rows_per_tile)],
                    hs.at[pl.ds(tbase, rows_per_tile)])
    plsc.subcore_barrier()

    for ip in range(iphases):
      cbase = wid * chunks + ip * cph
      pltpu.sync_copy(src_hbm.at[pl.ds(cbase, cph)], src_v)
      pltpu.sync_copy(dst_hbm.at[pl.ds(cbase, cph)], dst_v)
      for b in range(nbuf):
        pltpu.async_copy(hs.at[src_v.at[b]], rows_v.at[b], sems.at[b])

      def group(g, carry):
        for b in range(nbuf):
          j = g * nbuf + b
          pltpu.make_async_copy(hs.at[src_v.at[j]], rows_v.at[b],
                                sems.at[b]).wait()
          pltpu.sync_copy(rows_v.at[b], acc.at[dst_v.at[j]], add=True)
          jn = j + nbuf

          @pl.when(jn < cph)
          def _():
            pltpu.async_copy(hs.at[src_v.at[jn]], rows_v.at[b], sems.at[b])
        return carry

      lax.fori_loop(0, cph // nbuf, group, 0)
    plsc.subcore_barrier()
    base = (p * NC + cid) * (rows_per_tile * NS) + tbase
    pltpu.sync_copy(acc.at[pl.ds(tbase, rows_per_tile)],
                    out_hbm.at[pl.ds(base, rows_per_tile)])


def _make_msg_kernel(npad, d, chunks, nbuf, nsplit, iphases,
                     dtype=jnp.float32):
  rows_per_tile = npad // NS
  w = d // nsplit
  return functools.partial(
      pl.kernel,
      out_type=jax.ShapeDtypeStruct((nsplit * NC * npad, w), dtype),
      mesh=_mesh(),
      compiler_params=pltpu.CompilerParams(
          needs_layout_passes=False, use_tc_tiling_on_sc=False),
      scratch_types=[
          pltpu.VMEM((chunks // iphases, K), jnp.int32),
          pltpu.VMEM((chunks // iphases, K), jnp.int32),
          pltpu.VMEM((nbuf, K, w), dtype),
          pltpu.SemaphoreType.DMA((nbuf,)),
          pltpu.VMEM_SHARED((npad, w), dtype),
          pltpu.VMEM_SHARED((npad, w), dtype),
      ],
  )(functools.partial(_msg_body, chunks, rows_per_tile, nbuf, nsplit, iphases,
                      w))


def _mm_scale_body(x_ref, w_ref, dinv_ref, out_ref):
  h1 = jnp.dot(dinv_ref[...] * x_ref[...], w_ref[...],
               preferred_element_type=jnp.float32)
  out_ref[...] = h1.astype(jnp.bfloat16)


def _layer2_body(p0_ref, p1_ref, h1_ref, dinv_ref, b1_ref, w2_ref, out_ref):
  dinv = dinv_ref[...]
  agg = (p0_ref[...].astype(jnp.float32) + p1_ref[...].astype(jnp.float32) +
         h1_ref[...].astype(jnp.float32))
  h = jnp.maximum(dinv * agg + b1_ref[...], 0.0)
  out_ref[...] = dinv * jnp.dot(h, w2_ref[...],
                                preferred_element_type=jnp.float32)


def _final_body(c, p0_ref, p1_ref, h2_ref, dinv_ref, b2_ref, out_ref):
  agg = p0_ref[...] + p1_ref[...] + h2_ref[...]
  o = (dinv_ref[...] * agg + b2_ref[...])[:, :c]
  m = jnp.max(o, axis=1, keepdims=True)
  z = o - m
  lse = jnp.log(jnp.sum(jnp.exp(z), axis=1, keepdims=True))
  out_ref[...] = z - lse


def kernel(x, edge_index, W1, b1, W2, b2):
  n, f_in = x.shape
  hidden = W1.shape[1]
  c = W2.shape[1]
  e = edge_index.shape[1]

  npad = ((n + 2 * BLK) // (2 * BLK)) * (2 * BLK)
  chunks = -(-e // (NW * K))
  chunks = ((chunks + 7) // 8) * 8
  epad = NW * chunks * K
  epw = epad // NW
  cpad = ((c + L - 1) // L) * L

  src = jnp.concatenate([edge_index[0], jnp.zeros((epad - e,), jnp.int32)])
  dst = jnp.concatenate([edge_index[1],
                         jnp.full((epad - e,), n, jnp.int32)])
  src2d = src.reshape(epad // K, K)
  dst2d = dst.reshape(epad // K, K)
  xp = jnp.zeros((npad, f_in), x.dtype).at[:n].set(x)
  w2p = jnp.zeros((f_in, cpad), W2.dtype).at[:, :c].set(W2)
  b2p = jnp.zeros((cpad,), b2.dtype).at[:c].set(b2)

  dinv_sq = _make_deg_kernel(epad // NS)(dst, jnp.zeros((128, 128),
                                                        jnp.float32))
  dinv_col = dinv_sq.reshape(128 * 128)[:npad].reshape(npad, 1)

  grid = npad // BLK
  row_spec = pl.BlockSpec((BLK, 1), lambda i: (i, 0))

  h1 = pl.pallas_call(
      _mm_scale_body,
      grid=(grid,),
      in_specs=[
          pl.BlockSpec((BLK, f_in), lambda i: (i, 0)),
          pl.BlockSpec((f_in, hidden), lambda i: (0, 0)),
          row_spec,
      ],
      out_specs=pl.BlockSpec((BLK, hidden), lambda i: (i, 0)),
      out_shape=jax.ShapeDtypeStruct((npad, hidden), jnp.bfloat16),
  )(xp, W1, dinv_col)

  p1 = _make_msg_kernel(npad, hidden, chunks, 2, 1, 1, jnp.bfloat16)(
      h1, src2d, dst2d, jnp.zeros((K, hidden), jnp.bfloat16))

  def _reg(r, wd):
    return pl.BlockSpec((BLK, wd), lambda i, r=r: (r * grid + i, 0))

  h2 = pl.pallas_call(
      _layer2_body,
      grid=(grid,),
      in_specs=[
          _reg(0, hidden), _reg(1, hidden),
          pl.BlockSpec((BLK, hidden), lambda i: (i, 0)),
          row_spec,
          pl.BlockSpec((1, hidden), lambda i: (0, 0)),
          pl.BlockSpec((hidden, cpad), lambda i: (0, 0)),
      ],
      out_specs=pl.BlockSpec((BLK, cpad), lambda i: (i, 0)),
      out_shape=jax.ShapeDtypeStruct((npad, cpad), jnp.float32),
  )(p1, p1, h1, dinv_col, b1.reshape(1, hidden), w2p)

  p2 = _make_msg_kernel(npad, cpad, chunks, 4, 1, 1)(
      h2, src2d, dst2d, jnp.zeros((K, cpad), jnp.float32))

  out = pl.pallas_call(
      functools.partial(_final_body, c),
      grid=(grid,),
      in_specs=[
          _reg(0, cpad), _reg(1, cpad),
          pl.BlockSpec((BLK, cpad), lambda i: (i, 0)),
          row_spec,
          pl.BlockSpec((1, cpad), lambda i: (0, 0)),
      ],
      out_specs=pl.BlockSpec((BLK, c), lambda i: (i, 0)),
      out_shape=jax.ShapeDtypeStruct((npad, c), jnp.float32),
  )(p2, p2, h2, dinv_col, b2p.reshape(1, cpad))

  return out[:n]


# speedup vs baseline: 1.3716x; 1.0764x over previous
import functools

import jax
import jax.numpy as jnp
from jax import lax
from jax.experimental import pallas as pl
from jax.experimental.pallas import tpu as pltpu
from jax.experimental.pallas import tpu_sc as plsc

L = 16
NC, NS = 2, 16
NW = NC * NS
K = 128
BLK = 1024


def _mesh():
  return plsc.VectorSubcoreMesh(core_axis_name="c", subcore_axis_name="s")


def _rsqrt16(x):
  i = plsc.bitcast(x, jnp.int32)
  y = plsc.bitcast(jnp.int32(0x5F3759DF) - (i >> 1), jnp.float32)
  for _ in range(3):
    y = y * (1.5 - 0.5 * x * y * y)
  return y


def _deg_body(ept, dst_hbm, zeros_hbm, dinv_hbm, dst_v, hist_v, iota_v, deg_t,
              deg_s):
  cid = lax.axis_index("c")
  sid = lax.axis_index("s")
  sl = pl.ds(sid * 8, 8)
  iota16 = lax.iota(jnp.int32, L)
  for k in range(128 // L):
    iota_v[0, pl.ds(k * L, L)] = iota16 + k * L

  @pl.when(cid == 0)
  def _():
    pltpu.sync_copy(zeros_hbm, hist_v)
    pltpu.sync_copy(zeros_hbm.at[pl.ds(0, 8)], deg_s.at[sl])
    pltpu.sync_copy(dst_hbm.at[pl.ds(sid * ept, ept)], dst_v)
    ones = jnp.full((L,), 1.0, jnp.float32)

    def body(i, carry):
      for u in range(4):
        idx = dst_v[pl.ds((i * 4 + u) * L, L)]
        plsc.addupdate_scatter(hist_v, [idx >> 7, idx & 127], ones)
      return carry

    lax.fori_loop(0, ept // (L * 4), body, 0)

  plsc.subcore_barrier()

  @pl.when(cid == 0)
  def _():
    pltpu.sync_copy(hist_v, deg_s.at[iota_v.at[0]], add=True)

  plsc.subcore_barrier()

  @pl.when(cid == 0)
  def _():
    pltpu.sync_copy(deg_s.at[sl], deg_t)
    for r in range(8):
      for k in range(128 // L):
        cs = pl.ds(k * L, L)
        deg_t[r, cs] = _rsqrt16(1.0 + deg_t[r, cs])
    pltpu.sync_copy(deg_t, dinv_hbm.at[sl])


def _make_deg_kernel(ept):
  return functools.partial(
      pl.kernel,
      out_type=jax.ShapeDtypeStruct((128, 128), jnp.float32),
      mesh=_mesh(),
      compiler_params=pltpu.CompilerParams(needs_layout_passes=False),
      scratch_types=[
          pltpu.VMEM((ept,), jnp.int32),
          pltpu.VMEM((128, 128), jnp.float32),
          pltpu.VMEM((1, 128), jnp.int32),
          pltpu.VMEM((8, 128), jnp.float32),
          pltpu.VMEM_SHARED((128, 128), jnp.float32),
      ],
  )(functools.partial(_deg_body, ept))


def _msg_body(chunks, rows_per_tile, nbuf, nsplit, iphases, w, *refs):
  h_list = refs[:nsplit]
  (src_hbm, dst_hbm, zeros_hbm, out_hbm,
   src_v, dst_v, rows_v, sems, hs, acc) = refs[nsplit:]
  cid = lax.axis_index("c")
  sid = lax.axis_index("s")
  wid = sid * NC + cid
  tbase = sid * rows_per_tile
  cph = chunks // iphases

  for p in range(nsplit):
    for z in range(rows_per_tile // K):
      pltpu.sync_copy(zeros_hbm, acc.at[pl.ds(tbase + z * K, K)])
    pltpu.sync_copy(h_list[p].at[pl.ds(tbase, rows_per_tile)],
                    hs.at[pl.ds(tbase, rows_per_tile)])
    plsc.subcore_barrier()

    for ip in range(iphases):
      cbase = wid * chunks + ip * cph
      pltpu.sync_copy(src_hbm.at[pl.ds(cbase, cph)], src_v)
      pltpu.sync_copy(dst_hbm.at[pl.ds(cbase, cph)], dst_v)
      for b in range(nbuf):
        pltpu.async_copy(hs.at[src_v.at[b]], rows_v.at[b], sems.at[b])

      def group(g, carry):
        for b in range(nbuf):
          j = g * nbuf + b
          pltpu.make_async_copy(hs.at[src_v.at[j]], rows_v.at[b],
                                sems.at[b]).wait()
          pltpu.sync_copy(rows_v.at[b], acc.at[dst_v.at[j]], add=True)
          jn = j + nbuf

          @pl.when(jn < cph)
          def _():
            pltpu.async_copy(hs.at[src_v.at[jn]], rows_v.at[b], sems.at[b])
        return carry

      lax.fori_loop(0, cph // nbuf, group, 0)
    plsc.subcore_barrier()
    base = (p * NC + cid) * (rows_per_tile * NS) + tbase
    pltpu.sync_copy(acc.at[pl.ds(tbase, rows_per_tile)],
                    out_hbm.at[pl.ds(base, rows_per_tile)])


def _make_msg_kernel(npad, d, chunks, nbuf, nsplit, iphases,
                     dtype=jnp.float32):
  rows_per_tile = npad // NS
  w = d // nsplit
  return functools.partial(
      pl.kernel,
      out_type=jax.ShapeDtypeStruct((nsplit * NC * npad, w), dtype),
      mesh=_mesh(),
      compiler_params=pltpu.CompilerParams(
          needs_layout_passes=False, use_tc_tiling_on_sc=False),
      scratch_types=[
          pltpu.VMEM((chunks // iphases, K), jnp.int32),
          pltpu.VMEM((chunks // iphases, K), jnp.int32),
          pltpu.VMEM((nbuf, K, w), dtype),
          pltpu.SemaphoreType.DMA((nbuf,)),
          pltpu.VMEM_SHARED((npad, w), dtype),
          pltpu.VMEM_SHARED((npad, w), dtype),
      ],
  )(functools.partial(_msg_body, chunks, rows_per_tile, nbuf, nsplit, iphases,
                      w))


def _mm_scale_body(x_ref, w_ref, dinv_ref, out_ref):
  h1 = jnp.dot(dinv_ref[...] * x_ref[...], w_ref[...],
               preferred_element_type=jnp.float32)
  out_ref[...] = h1.astype(jnp.bfloat16)


def _layer2_body(p0_ref, p1_ref, h1_ref, dinv_ref, b1_ref, w2_ref, out_ref):
  dinv = dinv_ref[...]
  agg = (p0_ref[...].astype(jnp.float32) + p1_ref[...].astype(jnp.float32) +
         h1_ref[...].astype(jnp.float32))
  h = jnp.maximum(dinv * agg + b1_ref[...], 0.0)
  h2 = dinv * jnp.dot(h, w2_ref[...], preferred_element_type=jnp.float32)
  out_ref[...] = h2.astype(jnp.bfloat16)


def _final_body(c, p0_ref, p1_ref, h2_ref, dinv_ref, b2_ref, out_ref):
  agg = (p0_ref[...].astype(jnp.float32) + p1_ref[...].astype(jnp.float32) +
         h2_ref[...].astype(jnp.float32))
  o = (dinv_ref[...] * agg + b2_ref[...])[:, :c]
  m = jnp.max(o, axis=1, keepdims=True)
  z = o - m
  lse = jnp.log(jnp.sum(jnp.exp(z), axis=1, keepdims=True))
  out_ref[...] = z - lse


def kernel(x, edge_index, W1, b1, W2, b2):
  n, f_in = x.shape
  hidden = W1.shape[1]
  c = W2.shape[1]
  e = edge_index.shape[1]

  npad = ((n + 2 * BLK) // (2 * BLK)) * (2 * BLK)
  chunks = -(-e // (NW * K))
  chunks = ((chunks + 7) // 8) * 8
  epad = NW * chunks * K
  epw = epad // NW
  cpad = ((c + 63) // 64) * 64

  src = jnp.concatenate([edge_index[0], jnp.zeros((epad - e,), jnp.int32)])
  dst = jnp.concatenate([edge_index[1],
                         jnp.full((epad - e,), n, jnp.int32)])
  src2d = src.reshape(epad // K, K)
  dst2d = dst.reshape(epad // K, K)
  xp = jnp.zeros((npad, f_in), x.dtype).at[:n].set(x)
  w2p = jnp.zeros((f_in, cpad), W2.dtype).at[:, :c].set(W2)
  b2p = jnp.zeros((cpad,), b2.dtype).at[:c].set(b2)

  dinv_sq = _make_deg_kernel(epad // NS)(dst, jnp.zeros((128, 128),
                                                        jnp.float32))
  dinv_col = dinv_sq.reshape(128 * 128)[:npad].reshape(npad, 1)

  grid = npad // BLK
  row_spec = pl.BlockSpec((BLK, 1), lambda i: (i, 0))

  h1 = pl.pallas_call(
      _mm_scale_body,
      grid=(grid,),
      in_specs=[
          pl.BlockSpec((BLK, f_in), lambda i: (i, 0)),
          pl.BlockSpec((f_in, hidden), lambda i: (0, 0)),
          row_spec,
      ],
      out_specs=pl.BlockSpec((BLK, hidden), lambda i: (i, 0)),
      out_shape=jax.ShapeDtypeStruct((npad, hidden), jnp.bfloat16),
  )(xp, W1, dinv_col)

  p1 = _make_msg_kernel(npad, hidden, chunks, 2, 1, 1, jnp.bfloat16)(
      h1, src2d, dst2d, jnp.zeros((K, hidden), jnp.bfloat16))

  def _reg(r, wd):
    return pl.BlockSpec((BLK, wd), lambda i, r=r: (r * grid + i, 0))

  h2 = pl.pallas_call(
      _layer2_body,
      grid=(grid,),
      in_specs=[
          _reg(0, hidden), _reg(1, hidden),
          pl.BlockSpec((BLK, hidden), lambda i: (i, 0)),
          row_spec,
          pl.BlockSpec((1, hidden), lambda i: (0, 0)),
          pl.BlockSpec((hidden, cpad), lambda i: (0, 0)),
      ],
      out_specs=pl.BlockSpec((BLK, cpad), lambda i: (i, 0)),
      out_shape=jax.ShapeDtypeStruct((npad, cpad), jnp.bfloat16),
  )(p1, p1, h1, dinv_col, b1.reshape(1, hidden), w2p)

  p2 = _make_msg_kernel(npad, cpad, chunks, 4, 1, 1, jnp.bfloat16)(
      h2, src2d, dst2d, jnp.zeros((K, cpad), jnp.bfloat16))

  out = pl.pallas_call(
      functools.partial(_final_body, c),
      grid=(grid,),
      in_specs=[
          _reg(0, cpad), _reg(1, cpad),
          pl.BlockSpec((BLK, cpad), lambda i: (i, 0)),
          row_spec,
          pl.BlockSpec((1, cpad), lambda i: (0, 0)),
      ],
      out_specs=pl.BlockSpec((BLK, c), lambda i: (i, 0)),
      out_shape=jax.ShapeDtypeStruct((npad, c), jnp.float32),
  )(p2, p2, h2, dinv_col, b2p.reshape(1, cpad))

  return out[:n]


# speedup vs baseline: 1.3726x; 1.0008x over previous
import functools

import jax
import jax.numpy as jnp
from jax import lax
from jax.experimental import pallas as pl
from jax.experimental.pallas import tpu as pltpu
from jax.experimental.pallas import tpu_sc as plsc

L = 16
NC, NS = 2, 16
NW = NC * NS
K = 128
BLK = 1024


def _mesh():
  return plsc.VectorSubcoreMesh(core_axis_name="c", subcore_axis_name="s")


def _rsqrt16(x):
  i = plsc.bitcast(x, jnp.int32)
  y = plsc.bitcast(jnp.int32(0x5F3759DF) - (i >> 1), jnp.float32)
  for _ in range(3):
    y = y * (1.5 - 0.5 * x * y * y)
  return y


def _deg_body(ept, dst_hbm, zeros_hbm, dinv_hbm, dst_v, hist_v, iota_v, deg_t,
              deg_s):
  cid = lax.axis_index("c")
  sid = lax.axis_index("s")
  sl = pl.ds(sid * 8, 8)
  iota16 = lax.iota(jnp.int32, L)
  for k in range(128 // L):
    iota_v[0, pl.ds(k * L, L)] = iota16 + k * L

  @pl.when(cid == 0)
  def _():
    pltpu.sync_copy(zeros_hbm, hist_v)
    pltpu.sync_copy(zeros_hbm.at[pl.ds(0, 8)], deg_s.at[sl])
    pltpu.sync_copy(dst_hbm.at[pl.ds(sid * ept, ept)], dst_v)
    ones = jnp.full((L,), 1.0, jnp.float32)

    def body(i, carry):
      for u in range(4):
        idx = dst_v[pl.ds((i * 4 + u) * L, L)]
        plsc.addupdate_scatter(hist_v, [idx >> 7, idx & 127], ones)
      return carry

    lax.fori_loop(0, ept // (L * 4), body, 0)

  plsc.subcore_barrier()

  @pl.when(cid == 0)
  def _():
    pltpu.sync_copy(hist_v, deg_s.at[iota_v.at[0]], add=True)

  plsc.subcore_barrier()

  @pl.when(cid == 0)
  def _():
    pltpu.sync_copy(deg_s.at[sl], deg_t)
    for r in range(8):
      for k in range(128 // L):
        cs = pl.ds(k * L, L)
        deg_t[r, cs] = _rsqrt16(1.0 + deg_t[r, cs])
    pltpu.sync_copy(deg_t, dinv_hbm.at[sl])


def _make_deg_kernel(ept):
  return functools.partial(
      pl.kernel,
      out_type=jax.ShapeDtypeStruct((128, 128), jnp.float32),
      mesh=_mesh(),
      compiler_params=pltpu.CompilerParams(needs_layout_passes=False),
      scratch_types=[
          pltpu.VMEM((ept,), jnp.int32),
          pltpu.VMEM((128, 128), jnp.float32),
          pltpu.VMEM((1, 128), jnp.int32),
          pltpu.VMEM((8, 128), jnp.float32),
          pltpu.VMEM_SHARED((128, 128), jnp.float32),
      ],
  )(functools.partial(_deg_body, ept))


def _msg_body(chunks, rows_per_tile, nbuf, nsplit, iphases, w, *refs):
  h_list = refs[:nsplit]
  (src_hbm, dst_hbm, zeros_hbm, out_hbm,
   src_v, dst_v, rows_v, sems, hs, acc) = refs[nsplit:]
  cid = lax.axis_index("c")
  sid = lax.axis_index("s")
  wid = sid * NC + cid
  tbase = sid * rows_per_tile
  cph = chunks // iphases

  for p in range(nsplit):
    for z in range(rows_per_tile // K):
      pltpu.sync_copy(zeros_hbm, acc.at[pl.ds(tbase + z * K, K)])
    pltpu.sync_copy(h_list[p].at[pl.ds(tbase, rows_per_tile)],
                    hs.at[pl.ds(tbase, rows_per_tile)])
    plsc.subcore_barrier()

    for ip in range(iphases):
      cbase = wid * chunks + ip * cph
      pltpu.sync_copy(src_hbm.at[pl.ds(cbase, cph)], src_v)
      pltpu.sync_copy(dst_hbm.at[pl.ds(cbase, cph)], dst_v)
      for b in range(nbuf):
        pltpu.async_copy(hs.at[src_v.at[b]], rows_v.at[b], sems.at[b])

      def group(g, carry):
        for b in range(nbuf):
          j = g * nbuf + b
          pltpu.make_async_copy(hs.at[src_v.at[j]], rows_v.at[b],
                                sems.at[b]).wait()
          pltpu.sync_copy(rows_v.at[b], acc.at[dst_v.at[j]], add=True)
          jn = j + nbuf

          @pl.when(jn < cph)
          def _():
            pltpu.async_copy(hs.at[src_v.at[jn]], rows_v.at[b], sems.at[b])
        return carry

      lax.fori_loop(0, cph // nbuf, group, 0)
    plsc.subcore_barrier()
    base = (p * NC + cid) * (rows_per_tile * NS) + tbase
    pltpu.sync_copy(acc.at[pl.ds(tbase, rows_per_tile)],
                    out_hbm.at[pl.ds(base, rows_per_tile)])


def _make_msg_kernel(npad, d, chunks, nbuf, nsplit, iphases,
                     dtype=jnp.float32):
  rows_per_tile = npad // NS
  w = d // nsplit
  return functools.partial(
      pl.kernel,
      out_type=jax.ShapeDtypeStruct((nsplit * NC * npad, w), dtype),
      mesh=_mesh(),
      compiler_params=pltpu.CompilerParams(
          needs_layout_passes=False, use_tc_tiling_on_sc=False),
      scratch_types=[
          pltpu.VMEM((chunks // iphases, K), jnp.int32),
          pltpu.VMEM((chunks // iphases, K), jnp.int32),
          pltpu.VMEM((nbuf, K, w), dtype),
          pltpu.SemaphoreType.DMA((nbuf,)),
          pltpu.VMEM_SHARED((npad, w), dtype),
          pltpu.VMEM_SHARED((npad, w), dtype),
      ],
  )(functools.partial(_msg_body, chunks, rows_per_tile, nbuf, nsplit, iphases,
                      w))


def _mm_scale_body(x_ref, w_ref, dinv_ref, out_ref):
  h1 = jnp.dot(dinv_ref[...] * x_ref[...], w_ref[...],
               preferred_element_type=jnp.float32)
  out_ref[...] = h1.astype(jnp.bfloat16)


def _layer2_body(p0_ref, p1_ref, h1_ref, dinv_ref, b1_ref, w2_ref, out_ref):
  dinv = dinv_ref[...]
  agg = (p0_ref[...].astype(jnp.float32) + p1_ref[...].astype(jnp.float32) +
         h1_ref[...].astype(jnp.float32))
  h = jnp.maximum(dinv * agg + b1_ref[...], 0.0)
  h2 = dinv * jnp.dot(h, w2_ref[...], preferred_element_type=jnp.float32)
  out_ref[...] = h2.astype(jnp.bfloat16)


def _final_body(c, p0_ref, p1_ref, h2_ref, dinv_ref, b2_ref, out_ref):
  agg = (p0_ref[...].astype(jnp.float32) + p1_ref[...].astype(jnp.float32) +
         h2_ref[...].astype(jnp.float32))
  o = (dinv_ref[...] * agg + b2_ref[...])[:, :c]
  m = jnp.max(o, axis=1, keepdims=True)
  z = o - m
  lse = jnp.log(jnp.sum(jnp.exp(z), axis=1, keepdims=True))
  out_ref[...] = z - lse


def kernel(x, edge_index, W1, b1, W2, b2):
  n, f_in = x.shape
  hidden = W1.shape[1]
  c = W2.shape[1]
  e = edge_index.shape[1]

  npad = ((n + 2 * BLK) // (2 * BLK)) * (2 * BLK)
  chunks = -(-e // (NW * K))
  chunks = ((chunks + 7) // 8) * 8
  epad = NW * chunks * K
  cpad = ((c + 63) // 64) * 64

  src = jnp.concatenate([edge_index[0], jnp.zeros((epad - e,), jnp.int32)])
  dst = jnp.concatenate([edge_index[1],
                         jnp.full((epad - e,), n, jnp.int32)])
  src2d = src.reshape(epad // K, K)
  dst2d = dst.reshape(epad // K, K)
  xp = jnp.zeros((npad, f_in), x.dtype).at[:n].set(x)
  w2p = jnp.zeros((f_in, cpad), W2.dtype).at[:, :c].set(W2)
  b2p = jnp.zeros((cpad,), b2.dtype).at[:c].set(b2)

  dinv_sq = _make_deg_kernel(epad // NS)(dst, jnp.zeros((128, 128),
                                                        jnp.float32))
  dinv_col = dinv_sq.reshape(128 * 128)[:npad].reshape(npad, 1)

  grid = npad // BLK
  row_spec = pl.BlockSpec((BLK, 1), lambda i: (i, 0))

  h1 = pl.pallas_call(
      _mm_scale_body,
      grid=(grid,),
      in_specs=[
          pl.BlockSpec((BLK, f_in), lambda i: (i, 0)),
          pl.BlockSpec((f_in, hidden), lambda i: (0, 0)),
          row_spec,
      ],
      out_specs=pl.BlockSpec((BLK, hidden), lambda i: (i, 0)),
      out_shape=jax.ShapeDtypeStruct((npad, hidden), jnp.bfloat16),
  )(xp, W1, dinv_col)

  p1 = _make_msg_kernel(npad, hidden, chunks, 2, 1, 1, jnp.bfloat16)(
      h1, src2d, dst2d, jnp.zeros((K, hidden), jnp.bfloat16))

  def _reg(r, wd):
    return pl.BlockSpec((BLK, wd), lambda i, r=r: (r * grid + i, 0))

  h2 = pl.pallas_call(
      _layer2_body,
      grid=(grid,),
      in_specs=[
          _reg(0, hidden), _reg(1, hidden),
          pl.BlockSpec((BLK, hidden), lambda i: (i, 0)),
          row_spec,
          pl.BlockSpec((1, hidden), lambda i: (0, 0)),
          pl.BlockSpec((hidden, cpad), lambda i: (0, 0)),
      ],
      out_specs=pl.BlockSpec((BLK, cpad), lambda i: (i, 0)),
      out_shape=jax.ShapeDtypeStruct((npad, cpad), jnp.bfloat16),
  )(p1, p1, h1, dinv_col, b1.reshape(1, hidden), w2p)

  p2 = _make_msg_kernel(npad, cpad, chunks, 4, 1, 1, jnp.bfloat16)(
      h2, src2d, dst2d, jnp.zeros((K, cpad), jnp.bfloat16))

  out = pl.pallas_call(
      functools.partial(_final_body, c),
      grid=(grid,),
      in_specs=[
          _reg(0, cpad), _reg(1, cpad),
          pl.BlockSpec((BLK, cpad), lambda i: (i, 0)),
          row_spec,
          pl.BlockSpec((1, cpad), lambda i: (0, 0)),
      ],
      out_specs=pl.BlockSpec((BLK, c), lambda i: (i, 0)),
      out_shape=jax.ShapeDtypeStruct((npad, c), jnp.float32),
  )(p2, p2, h2, dinv_col, b2p.reshape(1, cpad))

  return out[:n]
